# baseline (speedup 1.0000x reference)
import functools

import jax
import jax.numpy as jnp
from jax import lax
from jax.experimental import pallas as pl
from jax.experimental.pallas import tpu as pltpu
from jax.experimental.pallas import tpu_sc as plsc

T = 2048
N = 1024
BH = 32

_NC, _NS = 2, 16
_NW = _NC * _NS
_B_PER_W = T // _NW


@functools.cache
def _make_sc_gather():
    mesh = plsc.VectorSubcoreMesh(core_axis_name="c", subcore_axis_name="s")

    npc = 4
    h = _B_PER_W // npc

    @functools.partial(
        pl.kernel,
        out_type=jax.ShapeDtypeStruct((T, N), jnp.float32),
        mesh=mesh,
        scratch_types=[
            pltpu.VMEM((_B_PER_W,), jnp.int32),
            pltpu.VMEM((npc, h, N), jnp.float32),
            pltpu.SemaphoreType.DMA((npc,)),
            pltpu.SemaphoreType.DMA((npc,)),
        ],
    )
    def _sc_gather(idx_hbm, table_hbm, out_hbm, idx_v, rows_v, sem_g, sem_w):
        wid = lax.axis_index("s") * _NC + lax.axis_index("c")
        base = wid * _B_PER_W
        pltpu.sync_copy(idx_hbm.at[pl.ds(base, _B_PER_W)], idx_v)
        gathers = [
            pltpu.async_copy(
                table_hbm.at[idx_v.at[pl.ds(p * h, h)]], rows_v.at[p],
                sem_g.at[p])
            for p in range(npc)
        ]
        writes = []
        for p in range(npc):
            gathers[p].wait()
            writes.append(pltpu.async_copy(
                rows_v.at[p], out_hbm.at[pl.ds(base + p * h, h)],
                sem_w.at[p]))
        for w in writes:
            w.wait()

    return _sc_gather


NCH = 64
CHR = 1024
NB = 5
NEP = 4
EPR = T // NEP


def _stream_body(x_hbm, e_hbm, w_hbm, b_hbm, o_hbm,
                 x_ring, o_ring, te_ref, e_pp, w_v, b_v,
                 sem_x, sem_o, sem_e, sem_c):
    cw = pltpu.make_async_copy(w_hbm, w_v, sem_c.at[0])
    cb = pltpu.make_async_copy(b_hbm, b_v, sem_c.at[1])
    cw.start()
    cb.start()
    for p in range(2):
        pltpu.make_async_copy(
            e_hbm.at[pl.ds(p * EPR, EPR)], e_pp.at[p], sem_e.at[p]).start()
    for s in range(NB):
        pltpu.make_async_copy(x_hbm.at[s], x_ring.at[s], sem_x.at[s]).start()
    cw.wait()
    cb.wait()
    for p in range(NEP):
        pltpu.make_async_copy(
            e_hbm.at[pl.ds(p * EPR, EPR)], e_pp.at[p % 2], sem_e.at[p % 2]).wait()
        te_ref[pl.ds(p * EPR, EPR), :] = (
            lax.dot_general(
                e_pp[p % 2].astype(jnp.bfloat16), w_v[...],
                (((1,), (1,)), ((), ())),
                preferred_element_type=jnp.float32,
            )
            + b_v[...]
        )
        if p + 2 < NEP:
            pltpu.make_async_copy(
                e_hbm.at[pl.ds((p + 2) * EPR, EPR)], e_pp.at[p % 2],
                sem_e.at[p % 2]).start()
    for c in range(NCH):
        s = c % NB
        if c >= NB:
            pltpu.make_async_copy(o_ring.at[s], o_hbm.at[c - NB], sem_o.at[s]).wait()
        pltpu.make_async_copy(x_hbm.at[c], x_ring.at[s], sem_x.at[s]).wait()
        off = (c % 2) * CHR
        o_ring[s] = x_ring[s] + te_ref[pl.ds(off, CHR), :]
        pltpu.make_async_copy(o_ring.at[s], o_hbm.at[c], sem_o.at[s]).start()
        if c + NB < NCH:
            pltpu.make_async_copy(
                x_hbm.at[c + NB], x_ring.at[s], sem_x.at[s]).start()
    for c in range(NCH - NB, NCH):
        s = c % NB
        pltpu.make_async_copy(o_ring.at[s], o_hbm.at[c], sem_o.at[s]).wait()


def kernel(x, t, emb, W, b):
    e = _make_sc_gather()(t, emb)
    x2 = x.reshape(NCH, CHR, N)
    out = pl.pallas_call(
        _stream_body,
        in_specs=[
            pl.BlockSpec(memory_space=pl.ANY),
            pl.BlockSpec(memory_space=pl.ANY),
            pl.BlockSpec(memory_space=pl.ANY),
            pl.BlockSpec(memory_space=pl.ANY),
        ],
        out_specs=pl.BlockSpec(memory_space=pl.ANY),
        out_shape=jax.ShapeDtypeStruct((NCH, CHR, N), jnp.float32),
        scratch_shapes=[
            pltpu.VMEM((NB, CHR, N), jnp.float32),
            pltpu.VMEM((NB, CHR, N), jnp.float32),
            pltpu.VMEM((T, N), jnp.float32),
            pltpu.VMEM((2, EPR, N), jnp.float32),
            pltpu.VMEM((N, N), jnp.bfloat16),
            pltpu.VMEM((1, N), jnp.float32),
            pltpu.SemaphoreType.DMA((NB,)),
            pltpu.SemaphoreType.DMA((NB,)),
            pltpu.SemaphoreType.DMA((2,)),
            pltpu.SemaphoreType.DMA((2,)),
        ],
    )(x2, e, W.astype(jnp.bfloat16), b.reshape(1, N))
    return out.reshape(x.shape)
